# Initial kernel scaffold; baseline (speedup 1.0000x reference)
import functools

import jax
import jax.numpy as jnp
from jax import lax
from jax.experimental import pallas as pl
from jax.experimental.pallas import tpu as pltpu
from jax.experimental.pallas import tpu_sc as plsc

N_NODES = 10000
D_NODE = 128
D_EDGE = 128
D_HID = 512
N_EDGES = 320000

_B = 128
_NCHUNK = N_EDGES // _B
_NW = 32
_FULL = _NCHUNK // _NW
_EXTRA = _NCHUNK - _FULL * _NW
_ROWS_PER_TILE = N_NODES // 16

_mesh = plsc.VectorSubcoreMesh(
    core_axis_name="c", subcore_axis_name="s", num_cores=2, num_subcores=16)


@functools.partial(
    pl.kernel,
    out_type=jax.ShapeDtypeStruct((2, N_NODES, D_EDGE), jnp.float32),
    mesh=_mesh,
    scratch_types=[
        pltpu.VMEM((_FULL + 1, _B), jnp.int32),
        pltpu.VMEM((_B, D_EDGE), jnp.float32),
        pltpu.VMEM((_B, D_EDGE), jnp.float32),
        pltpu.VMEM_SHARED((N_NODES, D_EDGE), jnp.float32),
        pltpu.SemaphoreType.DMA,
        pltpu.SemaphoreType.DMA,
    ],
)
def _sc_scatter(edge_hbm, idx_hbm, out_hbm, idxbuf, eb0, eb1, acc, sem0, sem1):
    c = lax.axis_index("c")
    s = lax.axis_index("s")
    wid = s * 2 + c

    def _zbody(i, carry):
        eb0[i // 8, pl.ds((i % 8) * 16, 16)] = jnp.zeros((16,), jnp.float32)
        return carry
    lax.fori_loop(0, _B * 8, _zbody, 0)

    row0 = s * _ROWS_PER_TILE
    for j in range(5):
        sz = min(_B, _ROWS_PER_TILE - j * _B)
        pltpu.sync_copy(eb0.at[pl.ds(0, sz)], acc.at[pl.ds(row0 + j * _B, sz)])

    pltpu.sync_copy(idx_hbm.at[pl.ds(wid * _FULL, _FULL)],
                    idxbuf.at[pl.ds(0, _FULL)])

    @pl.when(wid < _EXTRA)
    def _():
        pltpu.sync_copy(idx_hbm.at[pl.ds(_NW * _FULL + wid, 1)],
                        idxbuf.at[pl.ds(_FULL, 1)])

    plsc.subcore_barrier()

    first = wid * _FULL

    def _start(cid, buf, sem):
        pltpu.async_copy(edge_hbm.at[pl.ds(cid * _B, _B)], buf, sem)

    def _wait(buf, sem):
        pltpu.make_async_copy(edge_hbm.at[pl.ds(0, _B)], buf, sem).wait()

    def _scat(buf, jj):
        pltpu.sync_copy(buf, acc.at[idxbuf.at[jj]], add=True)

    _start(first, eb0, sem0)

    def _body(j, carry):
        _start(first + 2 * j + 1, eb1, sem1)
        _wait(eb0, sem0)
        _scat(eb0, 2 * j)
        _start(first + 2 * j + 2, eb0, sem0)
        _wait(eb1, sem1)
        _scat(eb1, 2 * j + 1)
        return carry
    lax.fori_loop(0, _FULL // 2 - 1, _body, 0)

    _start(first + _FULL - 1, eb1, sem1)
    _wait(eb0, sem0)
    _scat(eb0, _FULL - 2)
    _wait(eb1, sem1)
    _scat(eb1, _FULL - 1)

    @pl.when(wid < _EXTRA)
    def _():
        pltpu.sync_copy(edge_hbm.at[pl.ds((_NW * _FULL + wid) * _B, _B)], eb0)
        _scat(eb0, _FULL)

    plsc.subcore_barrier()

    pltpu.sync_copy(acc.at[pl.ds(row0, _ROWS_PER_TILE)],
                    out_hbm.at[c, pl.ds(row0, _ROWS_PER_TILE)])


def _mlp_body(nf_ref, agg_ref, w1a_ref, w1b_ref, b1_ref, w2_ref, b2_ref,
              g_ref, bt_ref, out_ref):
    nf = nf_ref[...]
    a = agg_ref[0] + agg_ref[1]
    h = (jnp.dot(nf, w1a_ref[...], preferred_element_type=jnp.float32)
         + jnp.dot(a, w1b_ref[...], preferred_element_type=jnp.float32)
         + b1_ref[...])
    h = h * jax.nn.sigmoid(h)
    h = jnp.dot(h, w2_ref[...], preferred_element_type=jnp.float32) + b2_ref[...]
    mu = jnp.mean(h, axis=-1, keepdims=True)
    var = jnp.mean((h - mu) ** 2, axis=-1, keepdims=True)
    h = (h - mu) * lax.rsqrt(var + 1e-5) * g_ref[...] + bt_ref[...]
    out_ref[...] = h + nf


_R = 2000


def _mlp(nf, agg, w1a, w1b, b1, w2, b2, gamma, beta):
    return pl.pallas_call(
        _mlp_body,
        grid=(N_NODES // _R,),
        in_specs=[
            pl.BlockSpec((_R, D_NODE), lambda i: (i, 0)),
            pl.BlockSpec((2, _R, D_EDGE), lambda i: (0, i, 0)),
            pl.BlockSpec((D_NODE, D_HID), lambda i: (0, 0)),
            pl.BlockSpec((D_EDGE, D_HID), lambda i: (0, 0)),
            pl.BlockSpec((1, D_HID), lambda i: (0, 0)),
            pl.BlockSpec((D_HID, D_NODE), lambda i: (0, 0)),
            pl.BlockSpec((1, D_NODE), lambda i: (0, 0)),
            pl.BlockSpec((1, D_NODE), lambda i: (0, 0)),
            pl.BlockSpec((1, D_NODE), lambda i: (0, 0)),
        ],
        out_specs=pl.BlockSpec((_R, D_NODE), lambda i: (i, 0)),
        out_shape=jax.ShapeDtypeStruct((N_NODES, D_NODE), jnp.float32),
    )(nf, agg, w1a, w1b, b1, w2, b2, gamma, beta)


def kernel(node_features, edge_features, src_indices, W1, b1, W2, b2,
           gamma, beta):
    idx2d = src_indices.reshape(_NCHUNK, _B).astype(jnp.int32)
    agg = _sc_scatter(edge_features, idx2d)
    return _mlp(node_features, agg, W1[:D_NODE], W1[D_NODE:],
                b1.reshape(1, -1), W2, b2.reshape(1, -1),
                gamma.reshape(1, -1), beta.reshape(1, -1))


# speedup vs baseline: 7.7155x; 7.7155x over previous
import functools

import jax
import jax.numpy as jnp
from jax import lax
from jax.experimental import pallas as pl
from jax.experimental.pallas import tpu as pltpu
from jax.experimental.pallas import tpu_sc as plsc

N_NODES = 10000
D_NODE = 128
D_EDGE = 128
D_HID = 512
N_EDGES = 320000

_B = 128
_NCHUNK = N_EDGES // _B
_NW = 32
_FULL = _NCHUNK // _NW
_EXTRA = _NCHUNK - _FULL * _NW
_ACC_ROWS = 10240
_ROWS_PER_TILE = _ACC_ROWS // 16

_mesh = plsc.VectorSubcoreMesh(
    core_axis_name="c", subcore_axis_name="s", num_cores=2, num_subcores=16)


@functools.partial(
    pl.kernel,
    out_type=jax.ShapeDtypeStruct((2, _ACC_ROWS, D_EDGE), jnp.float32),
    mesh=_mesh,
    scratch_types=[
        pltpu.VMEM((_FULL + 1, 1, _B), jnp.int32),
        pltpu.VMEM((_B, D_EDGE), jnp.float32),
        pltpu.VMEM((_B, D_EDGE), jnp.float32),
        pltpu.VMEM_SHARED((_ACC_ROWS, D_EDGE), jnp.float32),
        pltpu.SemaphoreType.DMA,
        pltpu.SemaphoreType.DMA,
    ],
)
def _sc_scatter(edge_hbm, idx_hbm, out_hbm, idxbuf, eb0, eb1, acc, sem0, sem1):
    c = lax.axis_index("c")
    s = lax.axis_index("s")
    wid = s * 2 + c

    def _zbody(i, carry):
        eb0[i // 8, pl.ds((i % 8) * 16, 16)] = jnp.zeros((16,), jnp.float32)
        return carry
    lax.fori_loop(0, _B * 8, _zbody, 0)

    row0 = s * _ROWS_PER_TILE
    for j in range(_ROWS_PER_TILE // _B):
        pltpu.sync_copy(eb0, acc.at[pl.ds(row0 + j * _B, _B)])

    pltpu.sync_copy(idx_hbm.at[pl.ds(wid * _FULL, _FULL)],
                    idxbuf.at[pl.ds(0, _FULL)])

    @pl.when(wid < _EXTRA)
    def _():
        pltpu.sync_copy(idx_hbm.at[pl.ds(_NW * _FULL + wid, 1)],
                        idxbuf.at[pl.ds(_FULL, 1)])

    plsc.subcore_barrier()

    first = wid * _FULL

    def _start(cid, buf, sem):
        pltpu.async_copy(edge_hbm.at[pl.ds(cid * _B, _B)], buf, sem)

    def _wait(buf, sem):
        pltpu.make_async_copy(edge_hbm.at[pl.ds(0, _B)], buf, sem).wait()

    def _scat(buf, jj):
        pltpu.sync_copy(buf, acc.at[idxbuf.at[jj, 0]], add=True)

    _start(first, eb0, sem0)

    def _body(j, carry):
        _start(first + 2 * j + 1, eb1, sem1)
        _wait(eb0, sem0)
        _scat(eb0, 2 * j)
        _start(first + 2 * j + 2, eb0, sem0)
        _wait(eb1, sem1)
        _scat(eb1, 2 * j + 1)
        return carry
    lax.fori_loop(0, _FULL // 2 - 1, _body, 0)

    _start(first + _FULL - 1, eb1, sem1)
    _wait(eb0, sem0)
    _scat(eb0, _FULL - 2)
    _wait(eb1, sem1)
    _scat(eb1, _FULL - 1)

    @pl.when(wid < _EXTRA)
    def _():
        pltpu.sync_copy(edge_hbm.at[pl.ds((_NW * _FULL + wid) * _B, _B)], eb0)
        _scat(eb0, _FULL)

    plsc.subcore_barrier()

    pltpu.sync_copy(acc.at[pl.ds(row0, _ROWS_PER_TILE)],
                    out_hbm.at[c, pl.ds(row0, _ROWS_PER_TILE)])


def _mlp_body(nf_ref, agg_ref, w1a_ref, w1b_ref, b1_ref, w2_ref, b2_ref,
              g_ref, bt_ref, out_ref):
    nf = nf_ref[...]
    a = agg_ref[0] + agg_ref[1]
    h = (jnp.dot(nf, w1a_ref[...], preferred_element_type=jnp.float32)
         + jnp.dot(a, w1b_ref[...], preferred_element_type=jnp.float32)
         + b1_ref[...])
    h = h * jax.nn.sigmoid(h)
    h = jnp.dot(h, w2_ref[...], preferred_element_type=jnp.float32) + b2_ref[...]
    mu = jnp.mean(h, axis=-1, keepdims=True)
    var = jnp.mean((h - mu) ** 2, axis=-1, keepdims=True)
    h = (h - mu) * lax.rsqrt(var + 1e-5) * g_ref[...] + bt_ref[...]
    out_ref[...] = h + nf


_R = 2048


def _mlp(nf, agg, w1a, w1b, b1, w2, b2, gamma, beta):
    return pl.pallas_call(
        _mlp_body,
        grid=(_ACC_ROWS // _R,),
        in_specs=[
            pl.BlockSpec((_R, D_NODE), lambda i: (i, 0)),
            pl.BlockSpec((2, _R, D_EDGE), lambda i: (0, i, 0)),
            pl.BlockSpec((D_NODE, D_HID), lambda i: (0, 0)),
            pl.BlockSpec((D_EDGE, D_HID), lambda i: (0, 0)),
            pl.BlockSpec((1, D_HID), lambda i: (0, 0)),
            pl.BlockSpec((D_HID, D_NODE), lambda i: (0, 0)),
            pl.BlockSpec((1, D_NODE), lambda i: (0, 0)),
            pl.BlockSpec((1, D_NODE), lambda i: (0, 0)),
            pl.BlockSpec((1, D_NODE), lambda i: (0, 0)),
        ],
        out_specs=pl.BlockSpec((_R, D_NODE), lambda i: (i, 0)),
        out_shape=jax.ShapeDtypeStruct((N_NODES, D_NODE), jnp.float32),
    )(nf, agg, w1a, w1b, b1, w2, b2, gamma, beta)


def kernel(node_features, edge_features, src_indices, W1, b1, W2, b2,
           gamma, beta):
    idx3d = src_indices.reshape(_NCHUNK, 1, _B).astype(jnp.int32)
    agg = _sc_scatter(edge_features, idx3d)
    return _mlp(node_features, agg, W1[:D_NODE], W1[D_NODE:],
                b1.reshape(1, -1), W2, b2.reshape(1, -1),
                gamma.reshape(1, -1), beta.reshape(1, -1))
